# Initial kernel scaffold; baseline (speedup 1.0000x reference)
import jax
import jax.numpy as jnp
from jax.experimental import pallas as pl


def kernel(logits, mask):
    raise NotImplementedError("write your pallas kernel here")


# speedup vs baseline: 4.5222x; 4.5222x over previous
import functools

import jax
import jax.numpy as jnp
from jax import lax
from jax.experimental import pallas as pl
from jax.experimental.pallas import tpu as pltpu
from jax.experimental.pallas import tpu_sc as plsc

B = 128
N = 32768
L = 16
NCHUNK = N // L
NBINS = 1024
NWORKERS = 32
ROWS_PER = B // NWORKERS

F32 = jnp.float32
VN = float(jnp.finfo(jnp.float32).min) / 4.0
BIG = float(jnp.finfo(jnp.float32).max)
NEG_INF = float("-inf")


def _splat(x):
    return jnp.broadcast_to(x, (L,))


def _vsum(v):
    return _splat(jnp.sum(v))


def _mesh():
    return plsc.VectorSubcoreMesh(core_axis_name="c", subcore_axis_name="s")


@functools.partial(
    pl.kernel,
    out_type=jax.ShapeDtypeStruct((B, N), jnp.float32),
    mesh=_mesh(),
    compiler_params=pltpu.CompilerParams(needs_layout_passes=False),
    scratch_types=[
        pltpu.VMEM((N,), jnp.float32),
        pltpu.VMEM((N,), jnp.float32),
        pltpu.VMEM((NBINS * L,), jnp.float32),
        pltpu.VMEM((NBINS * L,), jnp.float32),
    ],
)
def _sc_portfolio(logits_hbm, maskf_hbm, out_hbm, zbuf, mcand, hcnt, hsum):
    wid = lax.axis_index("s") * 2 + lax.axis_index("c")
    lane = lax.iota(jnp.int32, L)
    zeros = jnp.full((L,), 0.0, F32)
    ones = jnp.full((L,), 1.0, F32)

    def row_body(r, carry0):
        row = wid * ROWS_PER + r
        pltpu.sync_copy(logits_hbm.at[row], zbuf)
        pltpu.sync_copy(maskf_hbm.at[row], mcand)

        def p1(j, carry):
            vmax, vmin, vcnt = carry
            sl = pl.ds(j * L, L)
            v = zbuf[sl]
            m = mcand[sl]
            act = m > 0.0
            z = jnp.where(act, v, jnp.full((L,), VN, F32))
            zbuf[sl] = z
            vmax = jnp.maximum(vmax, z)
            vmin = jnp.minimum(vmin, jnp.where(act, v, jnp.full((L,), BIG, F32)))
            vcnt = vcnt + jnp.where(act, ones, zeros)
            return vmax, vmin, vcnt

        vmax, vmin, vcnt = lax.fori_loop(
            0, NCHUNK, p1,
            (jnp.full((L,), -BIG, F32), jnp.full((L,), BIG, F32), zeros))
        mx = _splat(jnp.max(vmax))
        mn = _splat(jnp.min(vmin))
        nact = _vsum(vcnt)

        thr_c = mx - 1.0

        def p2(j, off):
            sl = pl.ds(j * L, L)
            v = zbuf[sl]
            c = v > thr_c
            ci = jnp.where(c, jnp.full((L,), 1, jnp.int32),
                           jnp.full((L,), 0, jnp.int32))
            pos = plsc.cumsum(ci)
            idx = pos + (off - 1)
            plsc.store_scatter(mcand, [idx], v, mask=c)
            return off + jnp.sum(ci)

        nc = lax.fori_loop(0, NCHUNK, p2, jnp.int32(0))
        nch = (nc + (L - 1)) >> 4

        def cand_stats(tau):
            def cs(ch, carry):
                csum, ccnt = carry
                v = mcand[pl.ds(ch * L, L)]
                valid = (lane + ch * L) < nc
                a = jnp.logical_and(valid, v > tau)
                csum = csum + jnp.where(a, v, zeros)
                ccnt = ccnt + jnp.where(a, ones, zeros)
                return csum, ccnt
            s, c = lax.fori_loop(0, nch, cs, (zeros, zeros))
            return _vsum(s), _vsum(c)

        def newton_cond(st):
            _tau, done, it = st
            return jnp.logical_and(jnp.logical_not(done), it < 64)

        def newton_body(st):
            tau, done, it = st
            s, c = cand_stats(tau)
            t2 = (s - 1.0) / jnp.maximum(c, 1.0)
            return t2, jnp.all(t2 == tau), it + 1

        tau_star, _d, _i = lax.while_loop(
            newton_cond, newton_body,
            (mx - 1.0, jnp.bool_(False), jnp.int32(0)))
        _s_unused, kstd = cand_stats(tau_star)

        k = kstd + (float(N - 4) - nact)
        ireg = k - nact

        def finite_tau(_):
            def hz(bb, _c):
                sl = pl.ds(bb * L, L)
                hcnt[sl] = zeros
                hsum[sl] = zeros
                return 0
            lax.fori_loop(0, NBINS, hz, 0)

            w = jnp.where(mx > mn, (mx - mn) * (1.0 / float(NBINS)), ones)
            inv_w = 1.0 / w

            def hb(j, _c):
                v = zbuf[pl.ds(j * L, L)]
                bf = jnp.clip((v - mn) * inv_w, 0.0, float(NBINS - 1))
                bi = bf.astype(jnp.int32)
                idx = bi * L + lane
                plsc.addupdate_scatter(hcnt, [idx], ones)
                plsc.addupdate_scatter(hsum, [idx], v)
                return 0
            lax.fori_loop(0, NCHUNK, hb, 0)

            def sw1(t, carry):
                run, jbc = carry
                bb = (NBINS - 1) - t
                tb = _vsum(hcnt[pl.ds(bb * L, L)])
                run = run + tb
                jbc = jbc + jnp.where(run > k, ones, zeros)
                return run, jbc
            _run, jbf = lax.fori_loop(0, NBINS, sw1, (zeros, zeros))

            def sw2(t, carry):
                cab, sab = carry
                bb = (NBINS - 1) - t
                above = _splat(jnp.int32(bb)).astype(F32) > jbf
                tb = _vsum(hcnt[pl.ds(bb * L, L)])
                ts = _vsum(hsum[pl.ds(bb * L, L)])
                cab = cab + jnp.where(above, tb, zeros)
                sab = sab + jnp.where(above, ts, zeros)
                return cab, sab
            cab, sab = lax.fori_loop(0, NBINS, sw2, (zeros, zeros))

            t_edge = mn + jbf * w
            m_rem = k - cab
            sk = sab + m_rem * t_edge
            return (sk - 1.0) / k

        tau_fin = lax.cond(jnp.all(ireg < 0.5), finite_tau, lambda _: zeros, 0)
        tau_unif = (ireg * F32(VN) - 1.0) / k
        tau = jnp.where(ireg >= 4.5, jnp.full((L,), NEG_INF, F32),
                        jnp.where(ireg >= 0.5, tau_unif, tau_fin))

        def p4(j, acc):
            v = zbuf[pl.ds(j * L, L)]
            return acc + jnp.maximum(v - tau, 0.0)
        s1 = _vsum(lax.fori_loop(0, NCHUNK, p4, zeros))
        r1 = 1.0 / jnp.maximum(s1, 1e-12)

        def p5(j, acc):
            sl = pl.ds(j * L, L)
            v = zbuf[sl]
            p = jnp.maximum(v - tau, 0.0)
            wv = p * r1
            wv = jnp.where(wv < 1e-6, zeros, wv)
            zbuf[sl] = wv
            return acc + wv
        ws = _vsum(lax.fori_loop(0, NCHUNK, p5, zeros))
        r2 = 1.0 / jnp.maximum(ws, 1e-12)

        def p6(j, _c):
            sl = pl.ds(j * L, L)
            zbuf[sl] = zbuf[sl] * r2
            return 0
        lax.fori_loop(0, NCHUNK, p6, 0)
        pltpu.sync_copy(zbuf, out_hbm.at[row])
        return carry0

    lax.fori_loop(0, ROWS_PER, row_body, 0)


def kernel(logits, mask):
    maskf = mask.astype(jnp.float32)
    return _sc_portfolio(logits, maskf)


# speedup vs baseline: 7.0664x; 1.5626x over previous
import functools

import jax
import jax.numpy as jnp
from jax import lax
from jax.experimental import pallas as pl
from jax.experimental.pallas import tpu as pltpu
from jax.experimental.pallas import tpu_sc as plsc

B = 128
N = 32768
L = 16
NCHUNK = N // L
NBINS = 1024
NWORKERS = 32
ROWS_PER = B // NWORKERS
U = 8

F32 = jnp.float32
VN = float(jnp.finfo(jnp.float32).min) / 4.0
BIG = float(jnp.finfo(jnp.float32).max)
NEG_INF = float("-inf")
NAN = float("nan")
NACT_NAN_MAX = float(N // 2 - 5)


def _splat(x):
    return jnp.broadcast_to(x, (L,))


def _vsum(v):
    return _splat(jnp.sum(v))


def _mesh():
    return plsc.VectorSubcoreMesh(core_axis_name="c", subcore_axis_name="s")


@functools.partial(
    pl.kernel,
    out_type=jax.ShapeDtypeStruct((B, N), jnp.float32),
    mesh=_mesh(),
    compiler_params=pltpu.CompilerParams(needs_layout_passes=False),
    scratch_types=[
        pltpu.VMEM((N,), jnp.float32),
        pltpu.VMEM((N,), jnp.float32),
        pltpu.VMEM((NBINS * L,), jnp.float32),
        pltpu.VMEM((NBINS * L,), jnp.float32),
    ],
)
def _sc_portfolio(logits_hbm, maskf_hbm, out_hbm, zbuf, mcand, hcnt, hsum):
    wid = lax.axis_index("s") * 2 + lax.axis_index("c")
    lane = lax.iota(jnp.int32, L)
    zeros = jnp.full((L,), 0.0, F32)
    ones = jnp.full((L,), 1.0, F32)

    def row_body(r, carry0):
        row = wid * ROWS_PER + r
        pltpu.sync_copy(logits_hbm.at[row], zbuf)
        pltpu.sync_copy(maskf_hbm.at[row], mcand)

        def p1(jj, carry):
            vmax, vmin, vcnt = carry
            for u in range(U):
                sl = pl.ds((jj * U + u) * L, L)
                v = zbuf[sl]
                m = mcand[sl]
                act = m > 0.0
                z = jnp.where(act, v, jnp.full((L,), VN, F32))
                zbuf[sl] = z
                vmax = jnp.maximum(vmax, z)
                vmin = jnp.minimum(vmin,
                                   jnp.where(act, v, jnp.full((L,), BIG, F32)))
                vcnt = vcnt + jnp.where(act, ones, zeros)
            return vmax, vmin, vcnt

        vmax, vmin, vcnt = lax.fori_loop(
            0, NCHUNK // U, p1,
            (jnp.full((L,), -BIG, F32), jnp.full((L,), BIG, F32), zeros))
        mx = _splat(jnp.max(vmax))
        mn = _splat(jnp.min(vmin))
        nact = _vsum(vcnt)

        def nan_row(_):
            nanv = jnp.full((L,), NAN, F32)

            def pn(jj, _c):
                for u in range(U):
                    zbuf[pl.ds((jj * U + u) * L, L)] = nanv
                return 0
            lax.fori_loop(0, NCHUNK // U, pn, 0)
            return 0

        def full_row(_):
            thr_c = mx - 1.0

            def p2(jj, off):
                for u in range(U):
                    sl = pl.ds((jj * U + u) * L, L)
                    v = zbuf[sl]
                    c = v > thr_c
                    ci = jnp.where(c, jnp.full((L,), 1, jnp.int32),
                                   jnp.full((L,), 0, jnp.int32))
                    pos = plsc.cumsum(ci)
                    idx = pos + (off - 1)
                    plsc.store_scatter(mcand, [idx], v, mask=c)
                    off = off + jnp.sum(ci)
                return off

            nc = lax.fori_loop(0, NCHUNK // U, p2, jnp.int32(0))
            nch = (nc + (L - 1)) >> 4

            def cand_stats(tau):
                def cs(ch, carry):
                    csum, ccnt = carry
                    v = mcand[pl.ds(ch * L, L)]
                    valid = (lane + ch * L) < nc
                    a = jnp.logical_and(valid, v > tau)
                    csum = csum + jnp.where(a, v, zeros)
                    ccnt = ccnt + jnp.where(a, ones, zeros)
                    return csum, ccnt
                s, c = lax.fori_loop(0, nch, cs, (zeros, zeros))
                return _vsum(s), _vsum(c)

            def newton_cond(st):
                _tau, done, it = st
                return jnp.logical_and(jnp.logical_not(done), it < 64)

            def newton_body(st):
                tau, done, it = st
                s, c = cand_stats(tau)
                t2 = (s - 1.0) / jnp.maximum(c, 1.0)
                return t2, jnp.all(t2 == tau), it + 1

            tau_star, _d, _i = lax.while_loop(
                newton_cond, newton_body,
                (mx - 1.0, jnp.bool_(False), jnp.int32(0)))
            _s_unused, kstd = cand_stats(tau_star)

            k = kstd + (float(N - 4) - nact)
            ireg = k - nact

            def finite_tau(_a):
                def hz(jj, _c):
                    for u in range(U):
                        sl = pl.ds((jj * U + u) * L, L)
                        hcnt[sl] = zeros
                        hsum[sl] = zeros
                    return 0
                lax.fori_loop(0, NBINS // U, hz, 0)

                w = jnp.where(mx > mn, (mx - mn) * (1.0 / float(NBINS)), ones)
                inv_w = 1.0 / w

                def hb(jj, _c):
                    for u in range(U):
                        v = zbuf[pl.ds((jj * U + u) * L, L)]
                        bf = jnp.clip((v - mn) * inv_w, 0.0, float(NBINS - 1))
                        bi = bf.astype(jnp.int32)
                        idx = bi * L + lane
                        plsc.addupdate_scatter(hcnt, [idx], ones)
                        plsc.addupdate_scatter(hsum, [idx], v)
                    return 0
                lax.fori_loop(0, NCHUNK // U, hb, 0)

                def sw1(tt, carry):
                    run, jbc = carry
                    for u in range(U):
                        bb = (NBINS - 1) - (tt * U + u)
                        tb = _vsum(hcnt[pl.ds(bb * L, L)])
                        run = run + tb
                        jbc = jbc + jnp.where(run > k, ones, zeros)
                    return run, jbc
                _run, jbf = lax.fori_loop(0, NBINS // U, sw1, (zeros, zeros))

                def sw2(tt, carry):
                    cab, sab = carry
                    for u in range(U):
                        bb = (NBINS - 1) - (tt * U + u)
                        above = _splat(jnp.int32(bb)).astype(F32) > jbf
                        tb = _vsum(hcnt[pl.ds(bb * L, L)])
                        ts = _vsum(hsum[pl.ds(bb * L, L)])
                        cab = cab + jnp.where(above, tb, zeros)
                        sab = sab + jnp.where(above, ts, zeros)
                    return cab, sab
                cab, sab = lax.fori_loop(0, NBINS // U, sw2, (zeros, zeros))

                t_edge = mn + jbf * w
                m_rem = k - cab
                sk = sab + m_rem * t_edge
                return (sk - 1.0) / k

            tau_fin = lax.cond(jnp.all(ireg < 0.5), finite_tau,
                               lambda _a: zeros, 0)
            tau_unif = (ireg * F32(VN) - 1.0) / k
            tau = jnp.where(ireg >= 4.5, jnp.full((L,), NEG_INF, F32),
                            jnp.where(ireg >= 0.5, tau_unif, tau_fin))

            def p4(jj, acc):
                for u in range(U):
                    v = zbuf[pl.ds((jj * U + u) * L, L)]
                    acc = acc + jnp.maximum(v - tau, 0.0)
                return acc
            s1 = _vsum(lax.fori_loop(0, NCHUNK // U, p4, zeros))
            r1 = 1.0 / jnp.maximum(s1, 1e-12)

            def p5(jj, acc):
                for u in range(U):
                    sl = pl.ds((jj * U + u) * L, L)
                    v = zbuf[sl]
                    p = jnp.maximum(v - tau, 0.0)
                    wv = p * r1
                    wv = jnp.where(wv < 1e-6, zeros, wv)
                    zbuf[sl] = wv
                    acc = acc + wv
                return acc
            ws = _vsum(lax.fori_loop(0, NCHUNK // U, p5, zeros))
            r2 = 1.0 / jnp.maximum(ws, 1e-12)

            def p6(jj, _c):
                for u in range(U):
                    sl = pl.ds((jj * U + u) * L, L)
                    zbuf[sl] = zbuf[sl] * r2
                return 0
            lax.fori_loop(0, NCHUNK // U, p6, 0)
            return 0

        lax.cond(jnp.all(nact <= NACT_NAN_MAX), nan_row, full_row, 0)
        pltpu.sync_copy(zbuf, out_hbm.at[row])
        return carry0

    lax.fori_loop(0, ROWS_PER, row_body, 0)


def kernel(logits, mask):
    maskf = mask.astype(jnp.float32)
    return _sc_portfolio(logits, maskf)
